# baseline (speedup 1.0000x reference)
import jax
import jax.numpy as jnp
from jax.experimental import pallas as pl

BLOCK_ROWS = 25000


def _linear_kernel(x_ref, wt_ref, b_ref, o_ref):
    o_ref[...] = (
        jnp.dot(x_ref[...], wt_ref[...], preferred_element_type=jnp.float32)
        + b_ref[...]
    )


def kernel(x, W, b):
    n, hidden = x.shape
    out_dim = W.shape[0]
    wt = W.T
    b2 = b.reshape(1, out_dim)
    grid = (n // BLOCK_ROWS,)
    return pl.pallas_call(
        _linear_kernel,
        grid=grid,
        in_specs=[
            pl.BlockSpec((BLOCK_ROWS, hidden), lambda i: (i, 0)),
            pl.BlockSpec((hidden, out_dim), lambda i: (0, 0)),
            pl.BlockSpec((1, out_dim), lambda i: (0, 0)),
        ],
        out_specs=pl.BlockSpec((BLOCK_ROWS, out_dim), lambda i: (i, 0)),
        out_shape=jax.ShapeDtypeStruct((n, out_dim), jnp.float32),
    )(x, wt, b2)


# speedup vs baseline: 1.1168x; 1.1168x over previous
import jax
import jax.numpy as jnp
from jax.experimental import pallas as pl

BLOCK_ROWS = 20000


def _linear_kernel(x_ref, w_ref, b_ref, o_ref):
    o_ref[...] = (
        jax.lax.dot_general(
            x_ref[...],
            w_ref[...],
            (((1,), (1,)), ((), ())),
            preferred_element_type=jnp.float32,
        )
        + b_ref[...]
    )


def kernel(x, W, b):
    n, hidden = x.shape
    out_dim = W.shape[0]
    b2 = b.reshape(1, out_dim)
    grid = (n // BLOCK_ROWS,)
    return pl.pallas_call(
        _linear_kernel,
        grid=grid,
        in_specs=[
            pl.BlockSpec((BLOCK_ROWS, hidden), lambda i: (i, 0)),
            pl.BlockSpec((out_dim, hidden), lambda i: (0, 0)),
            pl.BlockSpec((1, out_dim), lambda i: (0, 0)),
        ],
        out_specs=pl.BlockSpec((BLOCK_ROWS, out_dim), lambda i: (i, 0)),
        out_shape=jax.ShapeDtypeStruct((n, out_dim), jnp.float32),
    )(x, W, b2)


# speedup vs baseline: 1.1188x; 1.0018x over previous
import jax
import jax.numpy as jnp
from jax.experimental import pallas as pl
from jax.experimental.pallas import tpu as pltpu

BLOCK_ROWS = 20000


def _linear_kernel(x_ref, w_ref, b_ref, o_ref):
    o_ref[...] = (
        jax.lax.dot_general(
            x_ref[...],
            w_ref[...],
            (((1,), (1,)), ((), ())),
            preferred_element_type=jnp.float32,
        )
        + b_ref[...]
    )


def kernel(x, W, b):
    n, hidden = x.shape
    out_dim = W.shape[0]
    b2 = b.reshape(1, out_dim)
    grid = (n // BLOCK_ROWS,)
    return pl.pallas_call(
        _linear_kernel,
        grid=grid,
        in_specs=[
            pl.BlockSpec((BLOCK_ROWS, hidden), lambda i: (i, 0)),
            pl.BlockSpec((out_dim, hidden), lambda i: (0, 0)),
            pl.BlockSpec((1, out_dim), lambda i: (0, 0)),
        ],
        out_specs=pl.BlockSpec((BLOCK_ROWS, out_dim), lambda i: (i, 0)),
        out_shape=jax.ShapeDtypeStruct((n, out_dim), jnp.float32),
        compiler_params=pltpu.CompilerParams(
            dimension_semantics=("parallel",),
        ),
    )(x, W, b2)
